# baseline (speedup 1.0000x reference)
import functools

import jax
import jax.numpy as jnp
from jax import lax
from jax.experimental import pallas as pl
from jax.experimental.pallas import tpu as pltpu
from jax.experimental.pallas import tpu_sc as plsc

N_REGIONS = 100000
N_PAIRS = 1000000
N_COMP = 128
B = 16384

NC = 2
NS = 16
NW = NC * NS
B_PER_W = B // NW
CHUNK = 128
N_CHUNKS = B_PER_W // CHUNK
L = 16
NBUF = 6


@functools.partial(
    pl.kernel,
    mesh=plsc.VectorSubcoreMesh(core_axis_name="c", subcore_axis_name="s"),
    compiler_params=pltpu.CompilerParams(
        disable_bounds_checks=True,
        disable_semaphore_checks=True,
        skip_device_barrier=True,
    ),
    out_type=(
        jax.ShapeDtypeStruct((B, N_COMP), jnp.float32),
        jax.ShapeDtypeStruct((B, N_COMP), jnp.float32),
    ),
    scratch_types=[
        pltpu.VMEM((B_PER_W,), jnp.int32),
        pltpu.VMEM((B_PER_W,), jnp.int32),
        pltpu.VMEM((B_PER_W,), jnp.int32),
        [pltpu.VMEM((CHUNK, N_COMP), jnp.float32) for _ in range(NBUF)],
        pltpu.SemaphoreType.DMA,
        [pltpu.SemaphoreType.DMA for _ in range(NBUF)],
        [pltpu.SemaphoreType.DMA for _ in range(NBUF)],
    ],
)
def _pca_gather(idx_hbm, pcat_hbm, proj_hbm, out_i_hbm, out_j_hbm,
                idx_v, ri_v, rj_v, bufs, psem, gsems, wsems):
    wid = lax.axis_index("s") * NC + lax.axis_index("c")
    base = wid * B_PER_W
    pi_hbm = pcat_hbm.at[pl.ds(0, N_PAIRS)]
    pj_hbm = pcat_hbm.at[pl.ds(N_PAIRS, N_PAIRS)]

    pltpu.sync_copy(idx_hbm.at[pl.ds(base, B_PER_W)], idx_v)

    pair_copies = []
    for src_hbm, dst in ((pi_hbm, ri_v), (pj_hbm, rj_v)):
        for c in range(N_CHUNKS):
            pair_copies.append(pltpu.async_copy(
                src_hbm.at[idx_v.at[pl.ds(c * CHUNK, CHUNK)]],
                dst.at[pl.ds(c * CHUNK, CHUNK)],
                psem,
            ))
    for cp in pair_copies:
        cp.wait()

    jobs = [(rv, c, out_hbm)
            for rv, out_hbm in ((ri_v, out_i_hbm), (rj_v, out_j_hbm))
            for c in range(N_CHUNKS)]

    def fire_gather(t):
        rv, c, _ = jobs[t]
        b = t % NBUF
        return pltpu.async_copy(
            proj_hbm.at[rv.at[pl.ds(c * CHUNK, CHUNK)]], bufs[b], gsems[b])

    def fire_writeback(t):
        _, c, out_hbm = jobs[t]
        b = t % NBUF
        return pltpu.async_copy(
            bufs[b], out_hbm.at[pl.ds(base + c * CHUNK, CHUNK)], wsems[b])

    n_jobs = len(jobs)
    gathers = [None] * n_jobs
    wbs = [None] * n_jobs
    for t in range(min(NBUF, n_jobs)):
        gathers[t] = fire_gather(t)
    for t in range(n_jobs):
        gathers[t].wait()
        wbs[t] = fire_writeback(t)
        u = t + NBUF
        if u < n_jobs:
            wbs[t].wait()
            gathers[u] = fire_gather(u)
    for t in range(max(0, n_jobs - NBUF), n_jobs):
        wbs[t].wait()


def kernel(x, expanded_idx, pair_table, cached_projection):
    del x
    pair_cat = jnp.minimum(
        jnp.concatenate([pair_table[:, 0], pair_table[:, 1]]), N_REGIONS - 1)
    return _pca_gather(expanded_idx, pair_cat, cached_projection)


# speedup vs baseline: 1.1062x; 1.1062x over previous
import functools

import jax
import jax.numpy as jnp
from jax import lax
from jax.experimental import pallas as pl
from jax.experimental.pallas import tpu as pltpu
from jax.experimental.pallas import tpu_sc as plsc

N_REGIONS = 100000
N_PAIRS = 1000000
N_COMP = 128
B = 16384

NC = 2
NS = 16
NW = NC * NS
B_PER_W = B // NW
CHUNK = 128
N_CHUNKS = B_PER_W // CHUNK
L = 16
NBUF = 6


@functools.partial(
    pl.kernel,
    mesh=plsc.VectorSubcoreMesh(core_axis_name="c", subcore_axis_name="s"),
    compiler_params=pltpu.CompilerParams(
        disable_bounds_checks=True,
        disable_semaphore_checks=True,
        skip_device_barrier=True,
    ),
    out_type=(
        jax.ShapeDtypeStruct((B, N_COMP), jnp.float32),
        jax.ShapeDtypeStruct((B, N_COMP), jnp.float32),
    ),
    scratch_types=[
        pltpu.VMEM((B_PER_W,), jnp.int32),
        pltpu.VMEM((B_PER_W,), jnp.int32),
        pltpu.VMEM((B_PER_W,), jnp.int32),
        [pltpu.VMEM((CHUNK, N_COMP), jnp.float32) for _ in range(NBUF)],
        pltpu.SemaphoreType.DMA,
        [pltpu.SemaphoreType.DMA for _ in range(NBUF)],
        [pltpu.SemaphoreType.DMA for _ in range(NBUF)],
    ],
)
def _pca_gather(idx_hbm, pi_hbm, pj_hbm, proj_hbm, out_i_hbm, out_j_hbm,
                idx_v, ri_v, rj_v, bufs, psem, gsems, wsems):
    wid = lax.axis_index("s") * NC + lax.axis_index("c")
    base = wid * B_PER_W

    pltpu.sync_copy(idx_hbm.at[pl.ds(base, B_PER_W)], idx_v)

    pair_copies = []
    for src_hbm, dst in ((pi_hbm, ri_v), (pj_hbm, rj_v)):
        for c in range(N_CHUNKS):
            pair_copies.append(pltpu.async_copy(
                src_hbm.at[idx_v.at[pl.ds(c * CHUNK, CHUNK)]],
                dst.at[pl.ds(c * CHUNK, CHUNK)],
                psem,
            ))
    for cp in pair_copies:
        cp.wait()

    jobs = [(rv, c, out_hbm)
            for rv, out_hbm in ((ri_v, out_i_hbm), (rj_v, out_j_hbm))
            for c in range(N_CHUNKS)]

    def fire_gather(t):
        rv, c, _ = jobs[t]
        b = t % NBUF
        return pltpu.async_copy(
            proj_hbm.at[rv.at[pl.ds(c * CHUNK, CHUNK)]], bufs[b], gsems[b])

    def fire_writeback(t):
        _, c, out_hbm = jobs[t]
        b = t % NBUF
        return pltpu.async_copy(
            bufs[b], out_hbm.at[pl.ds(base + c * CHUNK, CHUNK)], wsems[b])

    n_jobs = len(jobs)
    gathers = [None] * n_jobs
    wbs = [None] * n_jobs
    for t in range(min(NBUF, n_jobs)):
        gathers[t] = fire_gather(t)
    for t in range(n_jobs):
        gathers[t].wait()
        wbs[t] = fire_writeback(t)
        u = t + NBUF
        if u < n_jobs:
            wbs[t].wait()
            gathers[u] = fire_gather(u)
    for t in range(max(0, n_jobs - NBUF), n_jobs):
        wbs[t].wait()


def kernel(x, expanded_idx, pair_table, cached_projection):
    del x
    pair_i = jnp.minimum(pair_table[:, 0], N_REGIONS - 1)
    pair_j = jnp.minimum(pair_table[:, 1], N_REGIONS - 1)
    return _pca_gather(expanded_idx, pair_i, pair_j, cached_projection)
